# Initial kernel scaffold; baseline (speedup 1.0000x reference)
import jax
import jax.numpy as jnp
from jax.experimental import pallas as pl


def kernel(edge_embedding, edge_idx, batch, W, b):
    raise NotImplementedError("write your pallas kernel here")


# speedup vs baseline: 3.7979x; 3.7979x over previous
import jax
import jax.numpy as jnp
from jax.experimental import pallas as pl
from jax.experimental.pallas import tpu as pltpu

_N_NODES = 10000
_N_EDGES = 320000
_D = 192
_N_GRAPHS = 16
_OUT_DIM = 3

_BLOCK_E = 6400
_GRID = _N_EDGES // _BLOCK_E


def _body(idx_ref, batch_ref, eb_ref, W_ref, b_ref, out_ref, acc_ref):
    i = pl.program_id(0)

    bt = batch_ref[...]
    g_iota = jax.lax.broadcasted_iota(jnp.int32, (_N_GRAPHS, _N_NODES), 0)
    starts_lo = jnp.sum((bt < g_iota).astype(jnp.int32), axis=1, keepdims=True)
    starts_hi = jnp.sum((bt < g_iota + 1).astype(jnp.int32), axis=1, keepdims=True)

    idx = idx_ref[0]
    cmp_lo = (idx >= starts_lo).astype(jnp.float32)
    cmp_hi = (idx >= starts_hi).astype(jnp.float32)
    one_hot = cmp_lo - cmp_hi

    partial = jnp.dot(one_hot, eb_ref[...], preferred_element_type=jnp.float32)

    @pl.when(i == 0)
    def _():
        acc_ref[...] = jnp.zeros_like(acc_ref)

    acc_ref[...] += partial

    @pl.when(i == _GRID - 1)
    def _():
        counts = (starts_hi - starts_lo).astype(jnp.float32)
        pooled = acc_ref[...] / jnp.maximum(counts, 1.0)
        out_ref[...] = (
            jnp.dot(pooled, W_ref[...], preferred_element_type=jnp.float32)
            + b_ref[...]
        )


def kernel(edge_embedding, edge_idx, batch, W, b):
    idx3 = edge_idx.astype(jnp.int32).reshape(_GRID, 1, _BLOCK_E)
    batch2 = batch.astype(jnp.int32).reshape(1, _N_NODES)
    b2 = b.reshape(1, _OUT_DIM)
    return pl.pallas_call(
        _body,
        grid=(_GRID,),
        in_specs=[
            pl.BlockSpec((1, 1, _BLOCK_E), lambda i: (i, 0, 0)),
            pl.BlockSpec((1, _N_NODES), lambda i: (0, 0)),
            pl.BlockSpec((_BLOCK_E, _D), lambda i: (i, 0)),
            pl.BlockSpec((_D, _OUT_DIM), lambda i: (0, 0)),
            pl.BlockSpec((1, _OUT_DIM), lambda i: (0, 0)),
        ],
        out_specs=pl.BlockSpec((_N_GRAPHS, _OUT_DIM), lambda i: (0, 0)),
        out_shape=jax.ShapeDtypeStruct((_N_GRAPHS, _OUT_DIM), jnp.float32),
        scratch_shapes=[pltpu.VMEM((_N_GRAPHS, _D), jnp.float32)],
        compiler_params=pltpu.CompilerParams(
            dimension_semantics=("arbitrary",),
        ),
    )(idx3, batch2, edge_embedding, W, b2)


# speedup vs baseline: 3.7983x; 1.0001x over previous
import jax
import jax.numpy as jnp
from jax.experimental import pallas as pl
from jax.experimental.pallas import tpu as pltpu

_N_NODES = 10000
_N_EDGES = 320000
_D = 192
_N_GRAPHS = 16
_OUT_DIM = 3

_BLOCK_E = 16000
_GRID = _N_EDGES // _BLOCK_E


def _body(idx_ref, batch_ref, eb_ref, W_ref, b_ref, out_ref, acc_ref, st_ref):
    i = pl.program_id(0)

    @pl.when(i == 0)
    def _():
        bt = batch_ref[...]
        g_iota = jax.lax.broadcasted_iota(jnp.int32, (_N_GRAPHS, _N_NODES), 0)
        st_ref[:, 0:1] = jnp.sum(
            (bt < g_iota).astype(jnp.int32), axis=1, keepdims=True
        )
        st_ref[:, 1:2] = jnp.sum(
            (bt < g_iota + 1).astype(jnp.int32), axis=1, keepdims=True
        )
        acc_ref[...] = jnp.zeros_like(acc_ref)

    starts_lo = st_ref[:, 0:1]
    starts_hi = st_ref[:, 1:2]

    idx = idx_ref[0]
    cmp_lo = (idx >= starts_lo).astype(jnp.float32)
    cmp_hi = (idx >= starts_hi).astype(jnp.float32)
    one_hot = cmp_lo - cmp_hi

    acc_ref[...] += jnp.dot(
        one_hot, eb_ref[...], preferred_element_type=jnp.float32
    )

    @pl.when(i == _GRID - 1)
    def _():
        counts = (starts_hi - starts_lo).astype(jnp.float32)
        pooled = acc_ref[...] / jnp.maximum(counts, 1.0)
        out_ref[...] = (
            jnp.dot(pooled, W_ref[...], preferred_element_type=jnp.float32)
            + b_ref[...]
        )


def kernel(edge_embedding, edge_idx, batch, W, b):
    idx3 = edge_idx.astype(jnp.int32).reshape(_GRID, 1, _BLOCK_E)
    batch2 = batch.astype(jnp.int32).reshape(1, _N_NODES)
    b2 = b.reshape(1, _OUT_DIM)
    return pl.pallas_call(
        _body,
        grid=(_GRID,),
        in_specs=[
            pl.BlockSpec((1, 1, _BLOCK_E), lambda i: (i, 0, 0)),
            pl.BlockSpec((1, _N_NODES), lambda i: (0, 0)),
            pl.BlockSpec((_BLOCK_E, _D), lambda i: (i, 0)),
            pl.BlockSpec((_D, _OUT_DIM), lambda i: (0, 0)),
            pl.BlockSpec((1, _OUT_DIM), lambda i: (0, 0)),
        ],
        out_specs=pl.BlockSpec((_N_GRAPHS, _OUT_DIM), lambda i: (0, 0)),
        out_shape=jax.ShapeDtypeStruct((_N_GRAPHS, _OUT_DIM), jnp.float32),
        scratch_shapes=[
            pltpu.VMEM((_N_GRAPHS, _D), jnp.float32),
            pltpu.VMEM((_N_GRAPHS, 2), jnp.int32),
        ],
        compiler_params=pltpu.CompilerParams(
            dimension_semantics=("arbitrary",),
        ),
    )(idx3, batch2, edge_embedding, W, b2)
